# Initial kernel scaffold; baseline (speedup 1.0000x reference)
import jax
import jax.numpy as jnp
from jax.experimental import pallas as pl


def kernel(x, adj, W1, b1, W2, b2):
    raise NotImplementedError("write your pallas kernel here")


# speedup vs baseline: 1.0552x; 1.0552x over previous
import jax
import jax.numpy as jnp
from jax.experimental import pallas as pl
from jax.experimental.pallas import tpu as pltpu


def _pick_block_rows(n: int) -> int:
    for br in (512, 400, 320, 256, 200, 160, 128, 80, 64, 40, 32, 16, 8):
        if n % br == 0:
            return br
    return n


def _gcn_body(x_ref, adj_ref, w1_ref, b1_ref, w2_ref, b2_ref,
              out_ref, s1_ref, s2_ref):
    p = pl.program_id(0)
    j = pl.program_id(1)
    br = adj_ref.shape[0]

    @pl.when((p == 0) & (j == 0))
    def _():
        s1_ref[...] = jnp.dot(x_ref[...], w1_ref[...],
                              preferred_element_type=jnp.float32)

    @pl.when(p == 0)
    def _():
        h = jnp.dot(adj_ref[...], s1_ref[...],
                    preferred_element_type=jnp.float32)
        h = jnp.maximum(h + b1_ref[...], 0.0)
        s2_ref[pl.ds(j * br, br), :] = jnp.dot(
            h, w2_ref[...], preferred_element_type=jnp.float32)

    @pl.when(p == 1)
    def _():
        out_ref[...] = jnp.dot(adj_ref[...], s2_ref[...],
                               preferred_element_type=jnp.float32) + b2_ref[...]


def kernel(x, adj, W1, b1, W2, b2):
    n, nfeat = x.shape
    nhid = W1.shape[1]
    nout = W2.shape[1]
    br = _pick_block_rows(n)
    nb = n // br

    grid = (2, nb)
    out = pl.pallas_call(
        _gcn_body,
        grid=grid,
        in_specs=[
            pl.BlockSpec((n, nfeat), lambda p, j: (0, 0)),
            pl.BlockSpec((br, n), lambda p, j: (j, 0)),
            pl.BlockSpec((nfeat, nhid), lambda p, j: (0, 0)),
            pl.BlockSpec((1, nhid), lambda p, j: (0, 0)),
            pl.BlockSpec((nhid, nout), lambda p, j: (0, 0)),
            pl.BlockSpec((1, nout), lambda p, j: (0, 0)),
        ],
        out_specs=pl.BlockSpec((br, nout), lambda p, j: (p * j, 0)),
        out_shape=jax.ShapeDtypeStruct((n, nout), jnp.float32),
        scratch_shapes=[
            pltpu.VMEM((n, nhid), jnp.float32),
            pltpu.VMEM((n, nout), jnp.float32),
        ],
        compiler_params=pltpu.CompilerParams(
            dimension_semantics=("arbitrary", "arbitrary"),
            vmem_limit_bytes=110 * 1024 * 1024,
        ),
    )(x, adj, W1, b1.reshape(1, nhid), W2, b2.reshape(1, nout))
    return out
